# baseline (speedup 1.0000x reference)
import functools

import jax
import jax.numpy as jnp
from jax import lax
from jax.experimental import pallas as pl
from jax.experimental.pallas import tpu as pltpu
from jax.experimental.pallas import tpu_sc as plsc

D = 1024
L = 16
NSLICE = D // L
NC, NS = 2, 16
R = 8192
ROWS_PER_SC = R // NC
CH = 256
NCHS = ROWS_PER_SC // CH
TR = CH // NS
NBUF = 4


def _sc_body(actions_hbm, emb_hbm, out_a,
             emb_v, tb0, tb1,
             sh0, sh1, sh2, sh3,
             cb0, cb1,
             si0, si1, si2, si3, so0, so1, so2, so3):
    cid = lax.axis_index("c")
    sid = lax.axis_index("s")
    base = cid * ROWS_PER_SC

    tbufs = (tb0, tb1)
    cbs = (cb0, cb1)
    shs = (sh0, sh1, sh2, sh3)
    in_sems = (si0, si1, si2, si3)
    out_sems = (so0, so1, so2, so3)
    sl_me = pl.ds(sid * TR, TR)

    pltpu.sync_copy(emb_hbm, emb_v)

    def start_in(c, b):
        pltpu.make_async_copy(
            actions_hbm.at[pl.ds(base + c * CH, CH)], shs[b], in_sems[b]
        ).start()

    def wait_in(b):
        pltpu.make_async_copy(
            actions_hbm.at[pl.ds(base, CH)], shs[b], in_sems[b]
        ).wait()

    def start_out(c, b):
        pltpu.make_async_copy(
            shs[b], out_a.at[pl.ds(base + c * CH, CH)], out_sems[b]
        ).start()

    def wait_out(b):
        pltpu.make_async_copy(
            shs[b], out_a.at[pl.ds(base, CH)], out_sems[b]
        ).wait()

    def compute(tb):
        GJ = 8
        for g in range(NSLICE // GJ):
            embs = [emb_v[2, pl.ds((g * GJ + k) * L, L)] for k in range(GJ)]

            @plsc.parallel_loop(0, TR, unroll=2)
            def _row(r):
                for k in range(GJ):
                    sl = pl.ds((g * GJ + k) * L, L)
                    tb[r, sl] = tb[r, sl] + embs[k]

    @pl.when(sid == 0)
    def _():
        for b in range(NBUF):
            start_in(b, b)
        wait_in(0)
        wait_in(1)

    plsc.subcore_barrier()
    pltpu.make_async_copy(shs[0].at[sl_me], tbufs[0], cbs[0]).start()

    def quad_body(q, carry):
        for b in range(NBUF):
            c = q * NBUF + b
            t = b % 2
            tb, tbn = tbufs[t], tbufs[1 - t]

            pltpu.make_async_copy(shs[b].at[sl_me], tb, cbs[t]).wait()

            @pl.when(c + 1 < NCHS)
            def _():
                pltpu.make_async_copy(
                    shs[(b + 1) % NBUF].at[sl_me], tbn, cbs[1 - t]
                ).start()

            compute(tb)
            pltpu.sync_copy(tb, shs[b].at[sl_me])

            @pl.when(sid == 0)
            def _():
                @pl.when(c + 2 < NCHS)
                def _():
                    wait_in((b + 2) % NBUF)

            plsc.subcore_barrier()

            @pl.when(sid == 0)
            def _():
                start_out(c, b)

                @pl.when((c >= 1) & (c + 3 < NCHS))
                def _():
                    bn = (b + 3) % NBUF
                    wait_out(bn)
                    start_in(c + 3, bn)

        return carry

    lax.fori_loop(0, NCHS // NBUF, quad_body, 0)

    @pl.when(sid == 0)
    def _():
        for b in range(NBUF):
            wait_out(b)


def _tc_body(lang_ref, frames_ref, emb_ref, out_l, out_f):
    out_l[...] = lang_ref[...] + emb_ref[0, :][None, :]
    out_f[...] = frames_ref[...] + emb_ref[1, :][None, :]


def kernel(lang, frames, actions, emb_weight):
    B, S, Dm = lang.shape
    lf = lang.reshape(R, Dm)
    ff = frames.reshape(R, Dm)
    af = actions.reshape(R, Dm)
    f32 = jnp.float32

    mesh = plsc.VectorSubcoreMesh(core_axis_name="c", subcore_axis_name="s")
    sc_call = functools.partial(
        pl.kernel,
        mesh=mesh,
        out_type=jax.ShapeDtypeStruct((R, Dm), f32),
        scratch_types=[
            pltpu.VMEM((3, Dm), f32),
            pltpu.VMEM((TR, Dm), f32),
            pltpu.VMEM((TR, Dm), f32),
            pltpu.VMEM_SHARED((CH, Dm), f32),
            pltpu.VMEM_SHARED((CH, Dm), f32),
            pltpu.VMEM_SHARED((CH, Dm), f32),
            pltpu.VMEM_SHARED((CH, Dm), f32),
            pltpu.SemaphoreType.DMA,
            pltpu.SemaphoreType.DMA,
            pltpu.SemaphoreType.DMA,
            pltpu.SemaphoreType.DMA,
            pltpu.SemaphoreType.DMA,
            pltpu.SemaphoreType.DMA,
            pltpu.SemaphoreType.DMA,
            pltpu.SemaphoreType.DMA,
            pltpu.SemaphoreType.DMA,
            pltpu.SemaphoreType.DMA,
        ],
    )(_sc_body)
    out_a = sc_call(af, emb_weight)

    BR = 1024
    spec = pl.BlockSpec((BR, Dm), lambda i: (i, 0))
    emb_spec = pl.BlockSpec((3, Dm), lambda i: (0, 0))
    out_l, out_f = pl.pallas_call(
        _tc_body,
        grid=(R // BR,),
        in_specs=[spec, spec, emb_spec],
        out_specs=[spec, spec],
        out_shape=[jax.ShapeDtypeStruct((R, Dm), f32)] * 2,
    )(lf, ff, emb_weight)

    return (out_l.reshape(B, S, Dm), out_f.reshape(B, S, Dm),
            out_a.reshape(B, S, Dm))


# speedup vs baseline: 1.0713x; 1.0713x over previous
import functools

import jax
import jax.numpy as jnp
from jax import lax
from jax.experimental import pallas as pl
from jax.experimental.pallas import tpu as pltpu
from jax.experimental.pallas import tpu_sc as plsc

D = 1024
L = 16
NSLICE = D // L
NC, NS = 2, 16
NW = NC * NS
R = 8192
ROWS_PER_W = R // NW
CR = 32
NCH = ROWS_PER_W // CR


def _sc_body(actions_hbm, emb_hbm, out_a,
             emb_v, buf0, buf1,
             si0, si1, so0, so1):
    wid = lax.axis_index("s") * NC + lax.axis_index("c")
    base = wid * ROWS_PER_W

    pltpu.sync_copy(emb_hbm, emb_v)

    bufs = (buf0, buf1)
    in_sems = (si0, si1)
    out_sems = (so0, so1)

    def start_in(c, b):
        pltpu.make_async_copy(
            actions_hbm.at[pl.ds(base + c * CR, CR)], bufs[b], in_sems[b]
        ).start()

    def wait_in(b):
        pltpu.make_async_copy(
            actions_hbm.at[pl.ds(base, CR)], bufs[b], in_sems[b]
        ).wait()

    def start_out(c, b):
        pltpu.make_async_copy(
            bufs[b], out_a.at[pl.ds(base + c * CR, CR)], out_sems[b]
        ).start()

    def wait_out(b):
        pltpu.make_async_copy(
            bufs[b], out_a.at[pl.ds(base, CR)], out_sems[b]
        ).wait()

    def compute(b):
        buf = bufs[b]
        GJ = 8
        for g in range(NSLICE // GJ):
            embs = [emb_v[2, pl.ds((g * GJ + k) * L, L)] for k in range(GJ)]

            @plsc.parallel_loop(0, CR, unroll=2)
            def _row(r):
                for k in range(GJ):
                    sl = pl.ds((g * GJ + k) * L, L)
                    buf[r, sl] = buf[r, sl] + embs[k]

    start_in(0, 0)
    start_in(1, 1)

    def pair_body(p, carry):
        for b in range(2):
            c = 2 * p + b
            wait_in(b)
            compute(b)
            start_out(c, b)

            @pl.when(p + 1 < NCH // 2)
            def _():
                wait_out(b)
                start_in(c + 2, b)

        return carry

    lax.fori_loop(0, NCH // 2, pair_body, 0)
    wait_out(0)
    wait_out(1)


def _tc_body(lang_ref, frames_ref, emb_ref, out_l, out_f):
    out_l[...] = lang_ref[...] + emb_ref[0, :][None, :]
    out_f[...] = frames_ref[...] + emb_ref[1, :][None, :]


def kernel(lang, frames, actions, emb_weight):
    B, S, Dm = lang.shape
    lf = lang.reshape(R, Dm)
    ff = frames.reshape(R, Dm)
    af = actions.reshape(R, Dm)
    f32 = jnp.float32

    mesh = plsc.VectorSubcoreMesh(core_axis_name="c", subcore_axis_name="s")
    sc_call = functools.partial(
        pl.kernel,
        mesh=mesh,
        out_type=jax.ShapeDtypeStruct((R, Dm), f32),
        scratch_types=[
            pltpu.VMEM((3, Dm), f32),
            pltpu.VMEM((CR, Dm), f32),
            pltpu.VMEM((CR, Dm), f32),
            pltpu.SemaphoreType.DMA,
            pltpu.SemaphoreType.DMA,
            pltpu.SemaphoreType.DMA,
            pltpu.SemaphoreType.DMA,
        ],
    )(_sc_body)
    BR = 1024
    spec = pl.BlockSpec((BR, Dm), lambda i: (i, 0))
    emb_spec = pl.BlockSpec((3, Dm), lambda i: (0, 0))
    out_l, out_f = pl.pallas_call(
        _tc_body,
        grid=(R // BR,),
        in_specs=[spec, spec, emb_spec],
        out_specs=[spec, spec],
        out_shape=[jax.ShapeDtypeStruct((R, Dm), f32)] * 2,
    )(lf, ff, emb_weight)

    out_a = sc_call(af, emb_weight)

    return (out_l.reshape(B, S, Dm), out_f.reshape(B, S, Dm),
            out_a.reshape(B, S, Dm))
